# Initial kernel scaffold; baseline (speedup 1.0000x reference)
import jax
import jax.numpy as jnp
from jax.experimental import pallas as pl


def kernel(node_embedding, edges, W, b):
    raise NotImplementedError("write your pallas kernel here")


# speedup vs baseline: 22.9151x; 22.9151x over previous
import functools

import jax
import jax.numpy as jnp
from jax import lax
from jax.experimental import pallas as pl
from jax.experimental.pallas import tpu as pltpu
from jax.experimental.pallas import tpu_sc as plsc

_N_NODES = 10000
_N_EDGES = 320000
_D = 128

_info = plsc.get_sparse_core_info()
_NC = _info.num_cores
_NS = _info.num_subcores
_L = _info.num_lanes
_NW = _NC * _NS
_E_PER_W = _N_EDGES // _NW


def _tc_tables_body(x_ref, w1_ref, w2_ref, b_ref, s_ref, t_ref):
    x = x_ref[...]
    s_ref[...] = (
        jnp.dot(x, w1_ref[...], preferred_element_type=jnp.float32) + b_ref[0]
    )
    t_ref[...] = jnp.dot(x, w2_ref[...], preferred_element_type=jnp.float32)


def _make_tables(node_embedding, w1, w2, b):
    blk = 1000
    grid = (_N_NODES // blk,)
    s, t = pl.pallas_call(
        _tc_tables_body,
        grid=grid,
        in_specs=[
            pl.BlockSpec((blk, _D), lambda i: (i, 0)),
            pl.BlockSpec((_D, 1), lambda i: (0, 0)),
            pl.BlockSpec((_D, 1), lambda i: (0, 0)),
            pl.BlockSpec(memory_space=pltpu.SMEM),
        ],
        out_specs=[
            pl.BlockSpec((blk, 1), lambda i: (i, 0)),
            pl.BlockSpec((blk, 1), lambda i: (i, 0)),
        ],
        out_shape=[
            jax.ShapeDtypeStruct((_N_NODES, 1), jnp.float32),
            jax.ShapeDtypeStruct((_N_NODES, 1), jnp.float32),
        ],
    )(node_embedding, w1, w2, b)
    return s, t


_sc_mesh = plsc.VectorSubcoreMesh(core_axis_name="c", subcore_axis_name="s")


@functools.partial(
    pl.kernel,
    mesh=_sc_mesh,
    out_type=jax.ShapeDtypeStruct((_N_EDGES,), jnp.float32),
    compiler_params=pltpu.CompilerParams(needs_layout_passes=False),
    scratch_types=[
        pltpu.VMEM((_N_NODES,), jnp.float32),
        pltpu.VMEM((_N_NODES,), jnp.float32),
        pltpu.VMEM((_E_PER_W,), jnp.int32),
        pltpu.VMEM((_E_PER_W,), jnp.int32),
        pltpu.VMEM((_E_PER_W,), jnp.float32),
    ],
)
def _sc_edge_logits(s_hbm, t_hbm, src_hbm, dst_hbm, out_hbm,
                    s_v, t_v, src_v, dst_v, o_v):
    wid = lax.axis_index("s") * _NC + lax.axis_index("c")
    base = wid * _E_PER_W
    pltpu.sync_copy(s_hbm, s_v)
    pltpu.sync_copy(t_hbm, t_v)
    pltpu.sync_copy(src_hbm.at[pl.ds(base, _E_PER_W)], src_v)
    pltpu.sync_copy(dst_hbm.at[pl.ds(base, _E_PER_W)], dst_v)

    def body(i, carry):
        sl = pl.ds(i * _L, _L)
        gs = plsc.load_gather(s_v, [src_v[sl]])
        gt = plsc.load_gather(t_v, [dst_v[sl]])
        o_v[sl] = gs + gt
        return carry

    lax.fori_loop(0, _E_PER_W // _L, body, 0, unroll=4)
    pltpu.sync_copy(o_v, out_hbm.at[pl.ds(base, _E_PER_W)])


def kernel(node_embedding, edges, W, b):
    w1 = W[0, :_D].reshape(_D, 1)
    w2 = W[0, _D:].reshape(_D, 1)
    src = edges[:, 0].astype(jnp.int32)
    dst = edges[:, 1].astype(jnp.int32)
    s, t = _make_tables(node_embedding, w1, w2, b)
    out = _sc_edge_logits(s.reshape(-1), t.reshape(-1), src, dst)
    return out.reshape(_N_EDGES, 1)
